# Initial kernel scaffold; baseline (speedup 1.0000x reference)
import jax
import jax.numpy as jnp
from jax.experimental import pallas as pl


def kernel(r_raw, box):
    raise NotImplementedError("write your pallas kernel here")


# speedup vs baseline: 2.5503x; 2.5503x over previous
import functools

import numpy as np
import jax
import jax.numpy as jnp
from jax import lax
from jax.experimental import pallas as pl
from jax.experimental.pallas import tpu as pltpu
from jax.experimental.pallas import tpu_sc as plsc

DL = 10.0
SIGMA = 5.0
SIGMA_SQ_HALF = SIGMA ** 2 / 2.0
TWOPI = 2.0 * np.pi
TWOPI_SQ = (2.0 * np.pi) ** 2
K_SQ_MAX = (TWOPI / DL) ** 2
BOX_CONST = np.full((3,), 200.0, dtype=np.float32)

def _static_kpoints():
    nk = np.maximum((BOX_CONST / DL).astype(np.int32), 1)
    kx = np.arange(-int(nk[0]), int(nk[0]) + 1, dtype=np.int32)
    ky = np.arange(-int(nk[1]), int(nk[1]) + 1, dtype=np.int32)
    kz = np.arange(-int(nk[2]), int(nk[2]) + 1, dtype=np.int32)
    kxt = (kx.astype(np.float32) / BOX_CONST[0]) ** 2
    kyt = (ky.astype(np.float32) / BOX_CONST[1]) ** 2
    kzt = (kz.astype(np.float32) / BOX_CONST[2]) ** 2
    ksq = np.float32(TWOPI_SQ) * (
        kxt[:, None, None] + kyt[None, :, None] + kzt[None, None, :]
    )
    mask = (ksq <= np.float32(K_SQ_MAX)) & (ksq > 0)
    ix, iy, iz = np.nonzero(mask)
    return (
        kx[ix].astype(np.float32),
        ky[iy].astype(np.float32),
        kz[iz].astype(np.float32),
    )


_KXF, _KYF, _KZF = _static_kpoints()
N_SEL = _KXF.shape[0]

NUM_CORES = 2
NUM_SUBCORES = 16
LANES = 16
NW = NUM_CORES * NUM_SUBCORES

VECS_PER_W = -(-N_SEL // (NW * LANES))
CHUNK = VECS_PER_W * LANES
N_PAD = CHUNK * NW


def _pad(a, fill):
    out = np.full((N_PAD,), fill, dtype=np.float32)
    out[:N_SEL] = a
    return out


_KX_PAD = _pad(_KXF, 1.0)
_KY_PAD = _pad(_KYF, 0.0)
_KZ_PAD = _pad(_KZF, 0.0)

_KT_PAD = np.stack([_KX_PAD, _KY_PAD, _KZ_PAD], axis=-1).reshape(-1)


def _ewald_body(kt_hbm, kx_hbm, ky_hbm, kz_hbm, boxp_hbm, kv3_hbm, fac_hbm,
                kt_v, kx_v, ky_v, kz_v, boxp_v, kv3_v, fac_v):
    wid = lax.axis_index("s") * NUM_CORES + lax.axis_index("c")
    base = pl.multiple_of(wid * CHUNK, 8)
    base3 = pl.multiple_of(wid * (CHUNK * 3), 8)

    pltpu.sync_copy(kt_hbm.at[pl.ds(base3, CHUNK * 3)], kt_v)
    pltpu.sync_copy(kx_hbm.at[pl.ds(base, CHUNK)], kx_v)
    pltpu.sync_copy(ky_hbm.at[pl.ds(base, CHUNK)], ky_v)
    pltpu.sync_copy(kz_hbm.at[pl.ds(base, CHUNK)], kz_v)
    pltpu.sync_copy(boxp_hbm, boxp_v)

    pat = [TWOPI / boxp_v[pl.ds(t * LANES, LANES)] for t in range(3)]
    inv = [TWOPI / boxp_v[pl.ds((3 + t) * LANES, LANES)] for t in range(3)]

    for j in range(VECS_PER_W):
        for t in range(3):
            st = pl.ds(j * (3 * LANES) + t * LANES, LANES)
            kv3_v[st] = kt_v[st] * pat[t]
        s = pl.ds(j * LANES, LANES)
        vx = kx_v[s] * inv[0]
        vy = ky_v[s] * inv[1]
        vz = kz_v[s] * inv[2]
        ksq = vx * vx + vy * vy + vz * vz
        fac_v[s] = (TWOPI * jnp.exp(-SIGMA_SQ_HALF * ksq)) / ksq

    pltpu.sync_copy(kv3_v, kv3_hbm.at[pl.ds(base3, CHUNK * 3)])
    pltpu.sync_copy(fac_v, fac_hbm.at[pl.ds(base, CHUNK)])


@functools.cache
def _build_sc_call():
    return pl.kernel(
        _ewald_body,
        out_type=[
            jax.ShapeDtypeStruct((N_PAD * 3,), jnp.float32),
            jax.ShapeDtypeStruct((N_PAD,), jnp.float32),
        ],
        mesh=plsc.VectorSubcoreMesh(
            core_axis_name="c", subcore_axis_name="s",
            num_cores=NUM_CORES, num_subcores=NUM_SUBCORES,
        ),
        scratch_types=[
            pltpu.VMEM((CHUNK * 3,), jnp.float32),
            pltpu.VMEM((CHUNK,), jnp.float32),
            pltpu.VMEM((CHUNK,), jnp.float32),
            pltpu.VMEM((CHUNK,), jnp.float32),
            pltpu.VMEM((6 * LANES,), jnp.float32),
            pltpu.VMEM((CHUNK * 3,), jnp.float32),
            pltpu.VMEM((CHUNK,), jnp.float32),
        ],
    )


def kernel(r_raw, box):
    del r_raw
    boxf = box.astype(jnp.float32)
    box_periodic = jnp.tile(boxf, LANES)
    box_uniform = jnp.repeat(boxf, LANES, total_repeat_length=3 * LANES)
    boxp = jnp.concatenate([box_periodic, box_uniform])
    kv3, fac = _build_sc_call()(
        jnp.asarray(_KT_PAD), jnp.asarray(_KX_PAD), jnp.asarray(_KY_PAD),
        jnp.asarray(_KZ_PAD), boxp,
    )
    kvec = kv3.reshape(N_PAD, 3)[:N_SEL]
    factor = fac[:N_SEL]
    return (kvec, factor)


# speedup vs baseline: 3.1062x; 1.2180x over previous
import functools

import numpy as np
import jax
import jax.numpy as jnp
from jax import lax
from jax.experimental import pallas as pl
from jax.experimental.pallas import tpu as pltpu
from jax.experimental.pallas import tpu_sc as plsc

DL = 10.0
SIGMA = 5.0
SIGMA_SQ_HALF = SIGMA ** 2 / 2.0
TWOPI = 2.0 * np.pi
TWOPI_SQ = (2.0 * np.pi) ** 2
K_SQ_MAX = (TWOPI / DL) ** 2
BOX_CONST = np.full((3,), 200.0, dtype=np.float32)

def _static_kpoints():
    nk = np.maximum((BOX_CONST / DL).astype(np.int32), 1)
    kx = np.arange(-int(nk[0]), int(nk[0]) + 1, dtype=np.int32)
    ky = np.arange(-int(nk[1]), int(nk[1]) + 1, dtype=np.int32)
    kz = np.arange(-int(nk[2]), int(nk[2]) + 1, dtype=np.int32)
    kxt = (kx.astype(np.float32) / BOX_CONST[0]) ** 2
    kyt = (ky.astype(np.float32) / BOX_CONST[1]) ** 2
    kzt = (kz.astype(np.float32) / BOX_CONST[2]) ** 2
    ksq = np.float32(TWOPI_SQ) * (
        kxt[:, None, None] + kyt[None, :, None] + kzt[None, None, :]
    )
    mask = (ksq <= np.float32(K_SQ_MAX)) & (ksq > 0)
    ix, iy, iz = np.nonzero(mask)
    return (
        kx[ix].astype(np.float32),
        ky[iy].astype(np.float32),
        kz[iz].astype(np.float32),
    )


_KXF, _KYF, _KZF = _static_kpoints()
N_SEL = _KXF.shape[0]

NUM_CORES = 2
NUM_SUBCORES = 16
LANES = 16
NW = NUM_CORES * NUM_SUBCORES

VECS_PER_W = -(-N_SEL // (NW * LANES))
CHUNK = VECS_PER_W * LANES
N_PAD = CHUNK * NW
TAIL = N_SEL - (NW - 1) * CHUNK
PACK = 6 * CHUNK


def _pad(a, fill):
    out = np.full((N_PAD,), fill, dtype=np.float32)
    out[:N_SEL] = a
    return out


_KX_PAD = _pad(_KXF, 1.0)
_KY_PAD = _pad(_KYF, 0.0)
_KZ_PAD = _pad(_KZF, 0.0)
_KT_PAD = np.stack([_KX_PAD, _KY_PAD, _KZ_PAD], axis=-1).reshape(-1)

_PACKED = np.empty((NW, PACK), dtype=np.float32)
_PACKED[:, : 3 * CHUNK] = _KT_PAD.reshape(NW, 3 * CHUNK)
_PACKED[:, 3 * CHUNK : 4 * CHUNK] = _KX_PAD.reshape(NW, CHUNK)
_PACKED[:, 4 * CHUNK : 5 * CHUNK] = _KY_PAD.reshape(NW, CHUNK)
_PACKED[:, 5 * CHUNK :] = _KZ_PAD.reshape(NW, CHUNK)
_PACKED = _PACKED.reshape(-1)


def _ewald_body(pk_hbm, boxp_hbm, kv3_hbm, fac_hbm, pk_v, boxp_v, kv3_v, fac_v):
    wid = lax.axis_index("s") * NUM_CORES + lax.axis_index("c")
    base = pl.multiple_of(wid * CHUNK, 8)
    base3 = pl.multiple_of(wid * (CHUNK * 3), 8)

    pltpu.sync_copy(pk_hbm.at[pl.ds(pl.multiple_of(wid * PACK, 8), PACK)], pk_v)
    pltpu.sync_copy(boxp_hbm, boxp_v)

    pat = [TWOPI / boxp_v[pl.ds(t * LANES, LANES)] for t in range(3)]
    inv = [TWOPI / boxp_v[pl.ds((3 + t) * LANES, LANES)] for t in range(3)]

    for j in range(VECS_PER_W):
        for t in range(3):
            st = pl.ds(j * (3 * LANES) + t * LANES, LANES)
            kv3_v[st] = pk_v[st] * pat[t]
        s = pl.ds(j * LANES, LANES)
        vx = pk_v[pl.ds(3 * CHUNK + j * LANES, LANES)] * inv[0]
        vy = pk_v[pl.ds(4 * CHUNK + j * LANES, LANES)] * inv[1]
        vz = pk_v[pl.ds(5 * CHUNK + j * LANES, LANES)] * inv[2]
        ksq = vx * vx + vy * vy + vz * vz
        fac_v[s] = (TWOPI * jnp.exp(-SIGMA_SQ_HALF * ksq)) / ksq

    @pl.when(wid < NW - 1)
    def _full():
        pltpu.sync_copy(kv3_v, kv3_hbm.at[pl.ds(base3, CHUNK * 3)])
        pltpu.sync_copy(fac_v, fac_hbm.at[pl.ds(base, CHUNK)])

    @pl.when(wid == NW - 1)
    def _tail():
        pltpu.sync_copy(kv3_v.at[pl.ds(0, TAIL * 3)],
                        kv3_hbm.at[pl.ds(base3, TAIL * 3)])
        pltpu.sync_copy(fac_v.at[pl.ds(0, TAIL)],
                        fac_hbm.at[pl.ds(base, TAIL)])


@functools.cache
def _build_sc_call():
    return pl.kernel(
        _ewald_body,
        out_type=[
            jax.ShapeDtypeStruct((N_SEL * 3,), jnp.float32),
            jax.ShapeDtypeStruct((N_SEL,), jnp.float32),
        ],
        mesh=plsc.VectorSubcoreMesh(
            core_axis_name="c", subcore_axis_name="s",
            num_cores=NUM_CORES, num_subcores=NUM_SUBCORES,
        ),
        scratch_types=[
            pltpu.VMEM((PACK,), jnp.float32),
            pltpu.VMEM((6 * LANES,), jnp.float32),
            pltpu.VMEM((CHUNK * 3,), jnp.float32),
            pltpu.VMEM((CHUNK,), jnp.float32),
        ],
    )


def kernel(r_raw, box):
    del r_raw
    boxf = box.astype(jnp.float32)
    box_periodic = jnp.tile(boxf, LANES)
    box_uniform = jnp.repeat(boxf, LANES, total_repeat_length=3 * LANES)
    boxp = jnp.concatenate([box_periodic, box_uniform])
    kv3, factor = _build_sc_call()(jnp.asarray(_PACKED), boxp)
    return (kv3.reshape(N_SEL, 3), factor)


# speedup vs baseline: 4.0366x; 1.2995x over previous
import functools

import numpy as np
import jax
import jax.numpy as jnp
from jax import lax
from jax.experimental import pallas as pl
from jax.experimental.pallas import tpu as pltpu
from jax.experimental.pallas import tpu_sc as plsc

DL = 10.0
SIGMA = 5.0
SIGMA_SQ_HALF = SIGMA ** 2 / 2.0
TWOPI = 2.0 * np.pi
TWOPI_SQ = (2.0 * np.pi) ** 2
K_SQ_MAX = (TWOPI / DL) ** 2
BOX_CONST = np.full((3,), 200.0, dtype=np.float32)

def _static_kpoints():
    nk = np.maximum((BOX_CONST / DL).astype(np.int32), 1)
    kx = np.arange(-int(nk[0]), int(nk[0]) + 1, dtype=np.int32)
    ky = np.arange(-int(nk[1]), int(nk[1]) + 1, dtype=np.int32)
    kz = np.arange(-int(nk[2]), int(nk[2]) + 1, dtype=np.int32)
    kxt = (kx.astype(np.float32) / BOX_CONST[0]) ** 2
    kyt = (ky.astype(np.float32) / BOX_CONST[1]) ** 2
    kzt = (kz.astype(np.float32) / BOX_CONST[2]) ** 2
    ksq = np.float32(TWOPI_SQ) * (
        kxt[:, None, None] + kyt[None, :, None] + kzt[None, None, :]
    )
    mask = (ksq <= np.float32(K_SQ_MAX)) & (ksq > 0)
    ix, iy, iz = np.nonzero(mask)
    return (
        kx[ix].astype(np.float32),
        ky[iy].astype(np.float32),
        kz[iz].astype(np.float32),
    )


_KXF, _KYF, _KZF = _static_kpoints()
N_SEL = _KXF.shape[0]

NUM_CORES = 2
NUM_SUBCORES = 16
LANES = 16
NW = NUM_CORES * NUM_SUBCORES

VECS_PER_W = -(-N_SEL // (NW * LANES))
CHUNK = VECS_PER_W * LANES
N_PAD = CHUNK * NW
TAIL = N_SEL - (NW - 1) * CHUNK
PACK = 3 * CHUNK


def _pad(a, fill):
    out = np.full((N_PAD,), fill, dtype=np.float32)
    out[:N_SEL] = a
    return out


_KX_PAD = _pad(_KXF, 1.0)
_KY_PAD = _pad(_KYF, 0.0)
_KZ_PAD = _pad(_KZF, 0.0)

_PACKED = np.empty((NW, PACK), dtype=np.float32)
_PACKED[:, :CHUNK] = _KX_PAD.reshape(NW, CHUNK)
_PACKED[:, CHUNK : 2 * CHUNK] = _KY_PAD.reshape(NW, CHUNK)
_PACKED[:, 2 * CHUNK :] = _KZ_PAD.reshape(NW, CHUNK)
_PACKED = _PACKED.reshape(-1)

_K3_I8 = np.stack([_KXF, _KYF, _KZF], axis=-1).astype(np.int8)

_U = np.zeros((3, 3 * LANES), dtype=np.float32)
for _c in range(3):
    _U[_c, _c * LANES : (_c + 1) * LANES] = 1.0


def _fac_body(pk_hbm, boxu_hbm, fac_hbm, pk_v, boxu_v, fac_v):
    wid = lax.axis_index("s") * NUM_CORES + lax.axis_index("c")
    base = pl.multiple_of(wid * CHUNK, 8)

    pltpu.sync_copy(pk_hbm.at[pl.ds(pl.multiple_of(wid * PACK, 8), PACK)], pk_v)
    pltpu.sync_copy(boxu_hbm, boxu_v)

    inv = [TWOPI / boxu_v[pl.ds(t * LANES, LANES)] for t in range(3)]

    for j in range(VECS_PER_W):
        s = pl.ds(j * LANES, LANES)
        vx = pk_v[pl.ds(j * LANES, LANES)] * inv[0]
        vy = pk_v[pl.ds(CHUNK + j * LANES, LANES)] * inv[1]
        vz = pk_v[pl.ds(2 * CHUNK + j * LANES, LANES)] * inv[2]
        ksq = vx * vx + vy * vy + vz * vz
        fac_v[s] = (TWOPI * jnp.exp(-SIGMA_SQ_HALF * ksq)) / ksq

    @pl.when(wid < NW - 1)
    def _full():
        pltpu.sync_copy(fac_v, fac_hbm.at[pl.ds(base, CHUNK)])

    @pl.when(wid == NW - 1)
    def _tail():
        pltpu.sync_copy(fac_v.at[pl.ds(0, TAIL)], fac_hbm.at[pl.ds(base, TAIL)])


@functools.cache
def _build_fac_call():
    return pl.kernel(
        _fac_body,
        out_type=jax.ShapeDtypeStruct((N_SEL,), jnp.float32),
        mesh=plsc.VectorSubcoreMesh(
            core_axis_name="c", subcore_axis_name="s",
            num_cores=NUM_CORES, num_subcores=NUM_SUBCORES,
        ),
        scratch_types=[
            pltpu.VMEM((PACK,), jnp.float32),
            pltpu.VMEM((3 * LANES,), jnp.float32),
            pltpu.VMEM((CHUNK,), jnp.float32),
        ],
    )


_KV_BLOCK = 4160


def _kvec_body(k_ref, box_ref, out_ref):
    inv = TWOPI / box_ref[...]
    out_ref[...] = k_ref[...].astype(jnp.float32) * inv


@functools.cache
def _build_kvec_call():
    grid = -(-N_SEL // _KV_BLOCK)
    return pl.pallas_call(
        _kvec_body,
        out_shape=jax.ShapeDtypeStruct((N_SEL, 3), jnp.float32),
        grid=(grid,),
        in_specs=[
            pl.BlockSpec((_KV_BLOCK, 3), lambda i: (i, 0)),
            pl.BlockSpec((1, 3), lambda i: (0, 0)),
        ],
        out_specs=pl.BlockSpec((_KV_BLOCK, 3), lambda i: (i, 0)),
    )


def kernel(r_raw, box):
    del r_raw
    boxf = box.astype(jnp.float32)
    u = jnp.asarray(_U)
    boxu = boxf[0] * u[0] + boxf[1] * u[1] + boxf[2] * u[2]
    factor = _build_fac_call()(jnp.asarray(_PACKED), boxu)
    kvec = _build_kvec_call()(jnp.asarray(_K3_I8), boxf.reshape(1, 3))
    return (kvec, factor)


# speedup vs baseline: 5.8752x; 1.4555x over previous
import functools

import numpy as np
import jax
import jax.numpy as jnp
from jax import lax
from jax.experimental import pallas as pl
from jax.experimental.pallas import tpu as pltpu
from jax.experimental.pallas import tpu_sc as plsc

DL = 10.0
SIGMA = 5.0
SIGMA_SQ_HALF = SIGMA ** 2 / 2.0
TWOPI = 2.0 * np.pi
TWOPI_SQ = (2.0 * np.pi) ** 2
K_SQ_MAX = (TWOPI / DL) ** 2
BOX_CONST = np.full((3,), 200.0, dtype=np.float32)

def _static_kpoints():
    nk = np.maximum((BOX_CONST / DL).astype(np.int32), 1)
    kx = np.arange(-int(nk[0]), int(nk[0]) + 1, dtype=np.int32)
    ky = np.arange(-int(nk[1]), int(nk[1]) + 1, dtype=np.int32)
    kz = np.arange(-int(nk[2]), int(nk[2]) + 1, dtype=np.int32)
    kxt = (kx.astype(np.float32) / BOX_CONST[0]) ** 2
    kyt = (ky.astype(np.float32) / BOX_CONST[1]) ** 2
    kzt = (kz.astype(np.float32) / BOX_CONST[2]) ** 2
    ksq = np.float32(TWOPI_SQ) * (
        kxt[:, None, None] + kyt[None, :, None] + kzt[None, None, :]
    )
    mask = (ksq <= np.float32(K_SQ_MAX)) & (ksq > 0)
    ix, iy, iz = np.nonzero(mask)
    return (
        kx[ix].astype(np.float32),
        ky[iy].astype(np.float32),
        kz[iz].astype(np.float32),
    )


_KXF, _KYF, _KZF = _static_kpoints()
N_SEL = _KXF.shape[0]

NUM_CORES = 2
NUM_SUBCORES = 16
LANES = 16
NW = NUM_CORES * NUM_SUBCORES

VECS_PER_W = -(-N_SEL // (NW * LANES))
CHUNK = VECS_PER_W * LANES
N_PAD = CHUNK * NW
TAIL = N_SEL - (NW - 1) * CHUNK
PACK = 3 * CHUNK


def _pad(a, fill):
    out = np.full((N_PAD,), fill, dtype=np.float32)
    out[:N_SEL] = a
    return out


_KX_PAD = _pad(_KXF, 1.0)
_KY_PAD = _pad(_KYF, 0.0)
_KZ_PAD = _pad(_KZF, 0.0)

_PACKED = np.empty((NW, PACK), dtype=np.float32)
_PACKED[:, :CHUNK] = _KX_PAD.reshape(NW, CHUNK)
_PACKED[:, CHUNK : 2 * CHUNK] = _KY_PAD.reshape(NW, CHUNK)
_PACKED[:, 2 * CHUNK :] = _KZ_PAD.reshape(NW, CHUNK)
_PACKED = _PACKED.reshape(-1)

_U = np.zeros((3, 3 * LANES), dtype=np.float32)
for _c in range(3):
    _U[_c, _c * LANES : (_c + 1) * LANES] = 1.0


def _ewald_body(pk_hbm, boxu_hbm, vx_hbm, vy_hbm, vz_hbm, fac_hbm,
                pk_v, boxu_v, kv_v, fac_v):
    wid = lax.axis_index("s") * NUM_CORES + lax.axis_index("c")
    base = pl.multiple_of(wid * CHUNK, 8)

    pltpu.sync_copy(pk_hbm.at[pl.ds(pl.multiple_of(wid * PACK, 8), PACK)], pk_v)
    pltpu.sync_copy(boxu_hbm, boxu_v)

    inv = [TWOPI / boxu_v[pl.ds(t * LANES, LANES)] for t in range(3)]

    for j in range(VECS_PER_W):
        s = pl.ds(j * LANES, LANES)
        vx = pk_v[pl.ds(j * LANES, LANES)] * inv[0]
        vy = pk_v[pl.ds(CHUNK + j * LANES, LANES)] * inv[1]
        vz = pk_v[pl.ds(2 * CHUNK + j * LANES, LANES)] * inv[2]
        kv_v[pl.ds(j * LANES, LANES)] = vx
        kv_v[pl.ds(CHUNK + j * LANES, LANES)] = vy
        kv_v[pl.ds(2 * CHUNK + j * LANES, LANES)] = vz
        ksq = vx * vx + vy * vy + vz * vz
        fac_v[s] = (TWOPI * jnp.exp(-SIGMA_SQ_HALF * ksq)) / ksq

    outs = (vx_hbm, vy_hbm, vz_hbm)

    @pl.when(wid < NW - 1)
    def _full():
        for c in range(3):
            pltpu.sync_copy(kv_v.at[pl.ds(c * CHUNK, CHUNK)],
                            outs[c].at[pl.ds(base, CHUNK)])
        pltpu.sync_copy(fac_v, fac_hbm.at[pl.ds(base, CHUNK)])

    @pl.when(wid == NW - 1)
    def _tail():
        for c in range(3):
            pltpu.sync_copy(kv_v.at[pl.ds(c * CHUNK, TAIL)],
                            outs[c].at[pl.ds(base, TAIL)])
        pltpu.sync_copy(fac_v.at[pl.ds(0, TAIL)], fac_hbm.at[pl.ds(base, TAIL)])


@functools.cache
def _build_sc_call():
    return pl.kernel(
        _ewald_body,
        out_type=[
            jax.ShapeDtypeStruct((N_SEL,), jnp.float32),
            jax.ShapeDtypeStruct((N_SEL,), jnp.float32),
            jax.ShapeDtypeStruct((N_SEL,), jnp.float32),
            jax.ShapeDtypeStruct((N_SEL,), jnp.float32),
        ],
        mesh=plsc.VectorSubcoreMesh(
            core_axis_name="c", subcore_axis_name="s",
            num_cores=NUM_CORES, num_subcores=NUM_SUBCORES,
        ),
        scratch_types=[
            pltpu.VMEM((PACK,), jnp.float32),
            pltpu.VMEM((3 * LANES,), jnp.float32),
            pltpu.VMEM((PACK,), jnp.float32),
            pltpu.VMEM((CHUNK,), jnp.float32),
        ],
    )


def kernel(r_raw, box):
    del r_raw
    boxf = box.astype(jnp.float32)
    u = jnp.asarray(_U)
    boxu = boxf[0] * u[0] + boxf[1] * u[1] + boxf[2] * u[2]
    vx, vy, vz, factor = _build_sc_call()(jnp.asarray(_PACKED), boxu)
    return (jnp.stack((vx, vy, vz), axis=-1), factor)


# speedup vs baseline: 5.9879x; 1.0192x over previous
import functools

import numpy as np
import jax
import jax.numpy as jnp
from jax import lax
from jax.experimental import pallas as pl
from jax.experimental.pallas import tpu as pltpu
from jax.experimental.pallas import tpu_sc as plsc

DL = 10.0
SIGMA = 5.0
SIGMA_SQ_HALF = SIGMA ** 2 / 2.0
TWOPI = 2.0 * np.pi
TWOPI_SQ = (2.0 * np.pi) ** 2
K_SQ_MAX = (TWOPI / DL) ** 2
BOX_CONST = np.full((3,), 200.0, dtype=np.float32)

def _static_kpoints():
    nk = np.maximum((BOX_CONST / DL).astype(np.int32), 1)
    kx = np.arange(-int(nk[0]), int(nk[0]) + 1, dtype=np.int32)
    ky = np.arange(-int(nk[1]), int(nk[1]) + 1, dtype=np.int32)
    kz = np.arange(-int(nk[2]), int(nk[2]) + 1, dtype=np.int32)
    kxt = (kx.astype(np.float32) / BOX_CONST[0]) ** 2
    kyt = (ky.astype(np.float32) / BOX_CONST[1]) ** 2
    kzt = (kz.astype(np.float32) / BOX_CONST[2]) ** 2
    ksq = np.float32(TWOPI_SQ) * (
        kxt[:, None, None] + kyt[None, :, None] + kzt[None, None, :]
    )
    mask = (ksq <= np.float32(K_SQ_MAX)) & (ksq > 0)
    ix, iy, iz = np.nonzero(mask)
    return (
        kx[ix].astype(np.float32),
        ky[iy].astype(np.float32),
        kz[iz].astype(np.float32),
    )


_KXF, _KYF, _KZF = _static_kpoints()
N_SEL = _KXF.shape[0]

NUM_CORES = 2
NUM_SUBCORES = 16
LANES = 16
NW = NUM_CORES * NUM_SUBCORES

VECS_PER_W = -(-N_SEL // (NW * LANES))
CHUNK = VECS_PER_W * LANES
N_PAD = CHUNK * NW
TAIL = N_SEL - (NW - 1) * CHUNK
PACK = 3 * CHUNK


def _pad(a, fill):
    out = np.full((N_PAD,), fill, dtype=np.float32)
    out[:N_SEL] = a
    return out


_KX_PAD = _pad(_KXF, 1.0)
_KY_PAD = _pad(_KYF, 0.0)
_KZ_PAD = _pad(_KZF, 0.0)

_PACKED = np.empty((NW, PACK), dtype=np.float32)
_PACKED[:, :CHUNK] = _KX_PAD.reshape(NW, CHUNK)
_PACKED[:, CHUNK : 2 * CHUNK] = _KY_PAD.reshape(NW, CHUNK)
_PACKED[:, 2 * CHUNK :] = _KZ_PAD.reshape(NW, CHUNK)
_PACKED = _PACKED.reshape(-1)

def _ewald_body(pk_hbm, box_hbm, vx_hbm, vy_hbm, vz_hbm, fac_hbm,
                pk_v, box_v, kv_v, fac_v, sem):
    wid = lax.axis_index("s") * NUM_CORES + lax.axis_index("c")
    base = pl.multiple_of(wid * CHUNK, 8)

    in_cp = pltpu.make_async_copy(
        pk_hbm.at[pl.ds(pl.multiple_of(wid * PACK, 8), PACK)], pk_v, sem)
    box_cp = pltpu.make_async_copy(box_hbm, box_v, sem)
    in_cp.start()
    box_cp.start()
    box_cp.wait()
    in_cp.wait()

    inv = [TWOPI / box_v[pl.ds(t * LANES, LANES)] for t in range(3)]

    is_tail = wid == NW - 1
    outs = (vx_hbm, vy_hbm, vz_hbm)

    def _out_copies(n):
        cps = [pltpu.make_async_copy(kv_v.at[pl.ds(c * CHUNK, n)],
                                     outs[c].at[pl.ds(base, n)], sem)
               for c in range(3)]
        cps.append(pltpu.make_async_copy(fac_v.at[pl.ds(0, n)],
                                         fac_hbm.at[pl.ds(base, n)], sem))
        return cps

    for c in range(3):
        for j in range(VECS_PER_W):
            s = pl.ds(c * CHUNK + j * LANES, LANES)
            kv_v[s] = pk_v[s] * inv[c]

        @pl.when(jnp.logical_not(is_tail))
        def _start_full(c=c):
            _out_copies(CHUNK)[c].start()

        @pl.when(is_tail)
        def _start_tail(c=c):
            _out_copies(TAIL)[c].start()

    for j in range(VECS_PER_W):
        s = pl.ds(j * LANES, LANES)
        vx = kv_v[pl.ds(j * LANES, LANES)]
        vy = kv_v[pl.ds(CHUNK + j * LANES, LANES)]
        vz = kv_v[pl.ds(2 * CHUNK + j * LANES, LANES)]
        ksq = vx * vx + vy * vy + vz * vz
        fac_v[s] = (TWOPI * jnp.exp(-SIGMA_SQ_HALF * ksq)) / ksq

    @pl.when(jnp.logical_not(is_tail))
    def _finish_full():
        cps = _out_copies(CHUNK)
        cps[3].start()
        for cp in cps:
            cp.wait()

    @pl.when(is_tail)
    def _finish_tail():
        cps = _out_copies(TAIL)
        cps[3].start()
        for cp in cps:
            cp.wait()


@functools.cache
def _build_sc_call():
    return pl.kernel(
        _ewald_body,
        out_type=[
            jax.ShapeDtypeStruct((N_SEL,), jnp.float32),
            jax.ShapeDtypeStruct((N_SEL,), jnp.float32),
            jax.ShapeDtypeStruct((N_SEL,), jnp.float32),
            jax.ShapeDtypeStruct((N_SEL,), jnp.float32),
        ],
        mesh=plsc.VectorSubcoreMesh(
            core_axis_name="c", subcore_axis_name="s",
            num_cores=NUM_CORES, num_subcores=NUM_SUBCORES,
        ),
        scratch_types=[
            pltpu.VMEM((PACK,), jnp.float32),
            pltpu.VMEM((3 * LANES,), jnp.float32),
            pltpu.VMEM((PACK,), jnp.float32),
            pltpu.VMEM((CHUNK,), jnp.float32),
            pltpu.SemaphoreType.DMA,
        ],
    )


def kernel(r_raw, box):
    del r_raw
    boxu = jnp.repeat(box.astype(jnp.float32), LANES,
                      total_repeat_length=3 * LANES)
    vx, vy, vz, factor = _build_sc_call()(jnp.asarray(_PACKED), boxu)
    return (jnp.stack((vx, vy, vz), axis=-1), factor)


# speedup vs baseline: 6.7134x; 1.1212x over previous
import functools

import numpy as np
import jax
import jax.numpy as jnp
from jax import lax
from jax.experimental import pallas as pl
from jax.experimental.pallas import tpu as pltpu
from jax.experimental.pallas import tpu_sc as plsc

DL = 10.0
SIGMA = 5.0
SIGMA_SQ_HALF = SIGMA ** 2 / 2.0
TWOPI = 2.0 * np.pi
TWOPI_SQ = (2.0 * np.pi) ** 2
K_SQ_MAX = (TWOPI / DL) ** 2
BOX_CONST = np.full((3,), 200.0, dtype=np.float32)

def _static_kpoints():
    nk = np.maximum((BOX_CONST / DL).astype(np.int32), 1)
    kx = np.arange(-int(nk[0]), int(nk[0]) + 1, dtype=np.int32)
    ky = np.arange(-int(nk[1]), int(nk[1]) + 1, dtype=np.int32)
    kz = np.arange(-int(nk[2]), int(nk[2]) + 1, dtype=np.int32)
    kxt = (kx.astype(np.float32) / BOX_CONST[0]) ** 2
    kyt = (ky.astype(np.float32) / BOX_CONST[1]) ** 2
    kzt = (kz.astype(np.float32) / BOX_CONST[2]) ** 2
    ksq = np.float32(TWOPI_SQ) * (
        kxt[:, None, None] + kyt[None, :, None] + kzt[None, None, :]
    )
    mask = (ksq <= np.float32(K_SQ_MAX)) & (ksq > 0)
    ix, iy, iz = np.nonzero(mask)
    return (
        kx[ix].astype(np.float32),
        ky[iy].astype(np.float32),
        kz[iz].astype(np.float32),
    )


_KXF, _KYF, _KZF = _static_kpoints()
N_SEL = _KXF.shape[0]

NUM_CORES = 2
NUM_SUBCORES = 16
LANES = 16
NW = NUM_CORES * NUM_SUBCORES

VECS_PER_W = -(-N_SEL // (NW * LANES))
CHUNK = VECS_PER_W * LANES
N_PAD = CHUNK * NW
TAIL = N_SEL - (NW - 1) * CHUNK
PACK = 3 * CHUNK


def _pad(a, fill):
    out = np.full((N_PAD,), fill, dtype=np.float32)
    out[:N_SEL] = a
    return out


_KX_PAD = _pad(_KXF, 1.0)
_KY_PAD = _pad(_KYF, 0.0)
_KZ_PAD = _pad(_KZF, 0.0)

_PACKED = np.empty((NW, PACK), dtype=np.float32)
_PACKED[:, :CHUNK] = _KX_PAD.reshape(NW, CHUNK)
_PACKED[:, CHUNK : 2 * CHUNK] = _KY_PAD.reshape(NW, CHUNK)
_PACKED[:, 2 * CHUNK :] = _KZ_PAD.reshape(NW, CHUNK)
_PACKED = _PACKED.reshape(-1)

def _ewald_body(pk_hbm, box_hbm, vx_hbm, vy_hbm, vz_hbm, fac_hbm,
                pk_v, box_v, kv_v, fac_v, sem):
    wid = lax.axis_index("s") * NUM_CORES + lax.axis_index("c")
    base = pl.multiple_of(wid * CHUNK, 8)

    in_cp = pltpu.make_async_copy(
        pk_hbm.at[pl.ds(pl.multiple_of(wid * PACK, 8), PACK)], pk_v, sem)
    box_cp = pltpu.make_async_copy(box_hbm, box_v.at[pl.ds(0, 3)], sem)
    in_cp.start()
    box_cp.start()
    box_cp.wait()
    in_cp.wait()

    barr = box_v[...]
    inv = [TWOPI / jnp.broadcast_to(barr[t], (LANES,)) for t in range(3)]

    is_tail = wid == NW - 1
    outs = (vx_hbm, vy_hbm, vz_hbm)

    def _out_copies(n):
        cps = [pltpu.make_async_copy(kv_v.at[pl.ds(c * CHUNK, n)],
                                     outs[c].at[pl.ds(base, n)], sem)
               for c in range(3)]
        cps.append(pltpu.make_async_copy(fac_v.at[pl.ds(0, n)],
                                         fac_hbm.at[pl.ds(base, n)], sem))
        return cps

    for c in range(3):
        for j in range(VECS_PER_W):
            s = pl.ds(c * CHUNK + j * LANES, LANES)
            kv_v[s] = pk_v[s] * inv[c]

        @pl.when(jnp.logical_not(is_tail))
        def _start_full(c=c):
            _out_copies(CHUNK)[c].start()

        @pl.when(is_tail)
        def _start_tail(c=c):
            _out_copies(TAIL)[c].start()

    for j in range(VECS_PER_W):
        s = pl.ds(j * LANES, LANES)
        vx = kv_v[pl.ds(j * LANES, LANES)]
        vy = kv_v[pl.ds(CHUNK + j * LANES, LANES)]
        vz = kv_v[pl.ds(2 * CHUNK + j * LANES, LANES)]
        ksq = vx * vx + vy * vy + vz * vz
        fac_v[s] = (TWOPI * jnp.exp(-SIGMA_SQ_HALF * ksq)) / ksq

    @pl.when(jnp.logical_not(is_tail))
    def _finish_full():
        cps = _out_copies(CHUNK)
        cps[3].start()
        for cp in cps:
            cp.wait()

    @pl.when(is_tail)
    def _finish_tail():
        cps = _out_copies(TAIL)
        cps[3].start()
        for cp in cps:
            cp.wait()


@functools.cache
def _build_sc_call():
    return pl.kernel(
        _ewald_body,
        out_type=[
            jax.ShapeDtypeStruct((N_SEL,), jnp.float32),
            jax.ShapeDtypeStruct((N_SEL,), jnp.float32),
            jax.ShapeDtypeStruct((N_SEL,), jnp.float32),
            jax.ShapeDtypeStruct((N_SEL,), jnp.float32),
        ],
        mesh=plsc.VectorSubcoreMesh(
            core_axis_name="c", subcore_axis_name="s",
            num_cores=NUM_CORES, num_subcores=NUM_SUBCORES,
        ),
        scratch_types=[
            pltpu.VMEM((PACK,), jnp.float32),
            pltpu.VMEM((LANES,), jnp.float32),
            pltpu.VMEM((PACK,), jnp.float32),
            pltpu.VMEM((CHUNK,), jnp.float32),
            pltpu.SemaphoreType.DMA,
        ],
    )


def kernel(r_raw, box):
    del r_raw
    vx, vy, vz, factor = _build_sc_call()(
        jnp.asarray(_PACKED), box.astype(jnp.float32))
    return (jnp.stack((vx, vy, vz), axis=-1), factor)
